# baseline (speedup 1.0000x reference)
import dataclasses
import functools

import jax
import jax.numpy as jnp
from jax import lax
from jax.experimental import pallas as pl
from jax.experimental.pallas import tpu as pltpu
from jax.experimental.pallas import tpu_sc as plsc

N = 10000
E = 320000
H = 128

NC = 2
NS = 16
NW = NC * NS
LANES = 16
SLAB = H // NS

BATCH = 128
NP = 10240
IDXG = 16
NGRP = 2560 // IDXG
NIT = NGRP // 2
IDX_ROWS = 2560 + 2 * IDXG
SLABS = 32
SW = H // SLABS
STRIDE = SW + 1


def _make_agg(with_cnt):
  mesh = plsc.VectorSubcoreMesh(core_axis_name="c", subcore_axis_name="s")
  out_type = [jax.ShapeDtypeStruct((SLABS, NP * STRIDE), jnp.float32)]
  scratch = [
      pltpu.VMEM((IDXG, BATCH), jnp.int32),
      pltpu.VMEM((IDXG, BATCH), jnp.int32),
      pltpu.VMEM((IDXG, BATCH), jnp.int32),
      pltpu.VMEM((IDXG, BATCH), jnp.int32),
      pltpu.VMEM((NP * STRIDE,), jnp.float32),
      pltpu.VMEM((NP * STRIDE,), jnp.float32),
  ] + [pltpu.SemaphoreType.DMA for _ in range(4)]
  if with_cnt:
    out_type.append(jax.ShapeDtypeStruct((NP,), jnp.float32))
    scratch.append(pltpu.VMEM((NP,), jnp.float32))

  def body(xst_hbm, src_hbm, dst_hbm, z5_hbm, z1_hbm, *refs):
    if with_cnt:
      (out_hbm, cnt_hbm, srcA, dstA, srcB, dstB, table, acc,
       semAs, semAd, semBs, semBd, cacc) = refs
    else:
      (out_hbm, srcA, dstA, srcB, dstB, table, acc,
       semAs, semAd, semBs, semBd) = refs
      cnt_hbm = cacc = None
    slab = lax.axis_index("s") * NC + lax.axis_index("c")

    pltpu.sync_copy(z5_hbm, acc)
    if with_cnt:
      pltpu.sync_copy(z1_hbm, cacc)
    pltpu.sync_copy(xst_hbm.at[slab], table)

    cols = [jnp.full((LANES,), c, jnp.int32) for c in range(SW)]
    ones16 = jnp.ones((LANES,), jnp.float32)

    def accumulate(srcX, dstX, j):
      for k in range(BATCH // LANES):
        s16 = srcX.at[j, pl.ds(k * LANES, LANES)][...] * STRIDE
        d16 = dstX.at[j, pl.ds(k * LANES, LANES)][...]
        d16s = d16 * STRIDE
        for c in range(SW):
          v16 = plsc.load_gather(table, [s16 + cols[c]])
          plsc.addupdate_scatter(acc, [d16s + cols[c]], v16)
        if with_cnt:
          @pl.when(slab == 0)
          def _():
            plsc.addupdate_scatter(cacc, [d16], ones16)

    def start_idx(r, srcX, dstX, semX_s, semX_d):
      pltpu.async_copy(src_hbm.at[pl.ds(r, IDXG)], srcX, semX_s)
      pltpu.async_copy(dst_hbm.at[pl.ds(r, IDXG)], dstX, semX_d)

    def wait_idx(srcX, dstX, semX_s, semX_d):
      pltpu.make_async_copy(src_hbm.at[pl.ds(0, IDXG)], srcX, semX_s).wait()
      pltpu.make_async_copy(dst_hbm.at[pl.ds(0, IDXG)], dstX, semX_d).wait()

    start_idx(0, srcA, dstA, semAs, semAd)
    start_idx(IDXG, srcB, dstB, semBs, semBd)

    @pl.loop(0, NIT)
    def _(i):
      r0 = i * 2 * IDXG
      wait_idx(srcA, dstA, semAs, semAd)
      for j in range(IDXG):
        accumulate(srcA, dstA, j)
      start_idx(r0 + 2 * IDXG, srcA, dstA, semAs, semAd)
      wait_idx(srcB, dstB, semBs, semBd)
      for j in range(IDXG):
        accumulate(srcB, dstB, j)
      start_idx(r0 + 3 * IDXG, srcB, dstB, semBs, semBd)

    wait_idx(srcA, dstA, semAs, semAd)
    wait_idx(srcB, dstB, semBs, semBd)

    pltpu.sync_copy(acc, out_hbm.at[slab])
    if with_cnt:
      @pl.when(slab == 0)
      def _():
        pltpu.sync_copy(cacc, cnt_hbm)

  cp = pltpu.CompilerParams()
  if "needs_layout_passes" in pltpu.CompilerParams.__dataclass_fields__:
    cp = dataclasses.replace(cp, needs_layout_passes=False)
  if "use_tc_tiling_on_sc" in pltpu.CompilerParams.__dataclass_fields__:
    cp = dataclasses.replace(cp, use_tc_tiling_on_sc=False)
  return pl.kernel(body, out_type=out_type, mesh=mesh, scratch_types=scratch,
                   compiler_params=cp)


_agg_cnt = _make_agg(True)
_agg = _make_agg(False)


BM = 512
GRID = NP // BM
CROWS = BM // 128


def _inproj_body(x_ref, w_ref, b_ref, inp_ref, h_ref):
  z = jnp.dot(x_ref[...], w_ref[...],
              preferred_element_type=jnp.float32) + b_ref[...]
  inp_ref[...] = z
  h_ref[...] = jnp.maximum(z, 0.0)


_inproj = pl.pallas_call(
    _inproj_body,
    grid=(GRID,),
    in_specs=[
        pl.BlockSpec((BM, H), lambda i: (i, 0)),
        pl.BlockSpec((H, H), lambda i: (0, 0)),
        pl.BlockSpec((1, H), lambda i: (0, 0)),
    ],
    out_specs=[
        pl.BlockSpec((BM, H), lambda i: (i, 0)),
        pl.BlockSpec((BM, H), lambda i: (i, 0)),
    ],
    out_shape=[jax.ShapeDtypeStruct((NP, H), jnp.float32)] * 2,
)


def _combine_body(final, parts_ref, cnts_ref, h_ref, inp_ref, wl_ref, bl_ref,
                  wr_ref, o_ref):
  agg = parts_ref[...]
  cnt = cnts_ref[...]
  agg = agg / jnp.maximum(cnt, 1.0)
  z = (jnp.dot(agg, wl_ref[...], preferred_element_type=jnp.float32)
       + bl_ref[...]
       + jnp.dot(h_ref[...], wr_ref[...], preferred_element_type=jnp.float32))
  if final:
    m = jnp.max(z, axis=-1, keepdims=True)
    lse = jnp.log(jnp.sum(jnp.exp(z - m), axis=-1, keepdims=True)) + m
    o_ref[...] = z - lse
  else:
    o_ref[...] = jnp.maximum(z, 0.0) + 0.2 * inp_ref[...]


def _make_combine(final):
  return pl.pallas_call(
      functools.partial(_combine_body, final),
      grid=(GRID,),
      in_specs=[
          pl.BlockSpec((BM, H), lambda i: (i, 0)),
          pl.BlockSpec((BM, 1), lambda i: (i, 0)),
          pl.BlockSpec((BM, H), lambda i: (i, 0)),
          pl.BlockSpec((BM, H), lambda i: (i, 0)),
          pl.BlockSpec((H, H), lambda i: (0, 0)),
          pl.BlockSpec((1, H), lambda i: (0, 0)),
          pl.BlockSpec((H, H), lambda i: (0, 0)),
      ],
      out_specs=pl.BlockSpec((BM, H), lambda i: (i, 0)),
      out_shape=jax.ShapeDtypeStruct((NP, H), jnp.float32),
  )


_combine_mid = _make_combine(False)
_combine_final = _make_combine(True)


def kernel(x, edge_index, W_in, b_in, Wl0, bl0, Wr0, Wl1, bl1, Wr1,
           Wl2, bl2, Wr2):
  pad = IDX_ROWS * BATCH - E
  src = jnp.concatenate(
      [edge_index[0], jnp.zeros((pad,), jnp.int32)]).reshape(IDX_ROWS, BATCH)
  dst = jnp.concatenate(
      [edge_index[1], jnp.full((pad,), N, jnp.int32)]).reshape(IDX_ROWS, BATCH)
  x_pad = jnp.concatenate([x, jnp.zeros((NP - N, H), jnp.float32)])

  z5 = jnp.zeros((NP * STRIDE,), jnp.float32)
  z1 = jnp.zeros((NP,), jnp.float32)

  def slabbed(a):
    a = a.reshape(NP, SLABS, SW)
    a = jnp.pad(a, ((0, 0), (0, 0), (0, STRIDE - SW)))
    return a.transpose(1, 0, 2).reshape(SLABS, NP * STRIDE)

  def unslabbed(p):
    p = p.reshape(SLABS, NP, STRIDE)[:, :, :SW]
    return p.transpose(1, 0, 2).reshape(NP, H)

  inp, h = _inproj(x_pad, W_in.T, b_in.reshape(1, H))

  parts, cnts = _agg_cnt(slabbed(h), src, dst, z5, z1)
  cnt3 = cnts.reshape(NP, 1)
  h = _combine_mid(unslabbed(parts), cnt3, h, inp, Wl0.T,
                   bl0.reshape(1, H), Wr0.T)
  parts, = _agg(slabbed(h), src, dst, z5, z1)
  h = _combine_mid(unslabbed(parts), cnt3, h, inp, Wl1.T,
                   bl1.reshape(1, H), Wr1.T)
  parts, = _agg(slabbed(h), src, dst, z5, z1)
  out = _combine_final(unslabbed(parts), cnt3, h, inp, Wl2.T,
                       bl2.reshape(1, H), Wr2.T)
  return out[:N]


# speedup vs baseline: 1.0315x; 1.0315x over previous
import dataclasses
import functools

import jax
import jax.numpy as jnp
from jax import lax
from jax.experimental import pallas as pl
from jax.experimental.pallas import tpu as pltpu
from jax.experimental.pallas import tpu_sc as plsc

N = 10000
E = 320000
H = 128

NC = 2
NS = 16
NW = NC * NS
LANES = 16
SLAB = H // NS

BATCH = 128
NP = 10240
IDXG = 8
NGRP = 2560 // IDXG
NIT = NGRP // 2
IDX_ROWS = 2560 + 2 * IDXG
SLABS = 32
SW = H // SLABS
STRIDE = SW + 1


def _make_agg(with_cnt):
  mesh = plsc.VectorSubcoreMesh(core_axis_name="c", subcore_axis_name="s")
  out_type = [jax.ShapeDtypeStruct((SLABS, NP * STRIDE), jnp.float32)]
  scratch = [
      pltpu.VMEM((IDXG, BATCH), jnp.int32),
      pltpu.VMEM((IDXG, BATCH), jnp.int32),
      pltpu.VMEM((IDXG, BATCH), jnp.int32),
      pltpu.VMEM((IDXG, BATCH), jnp.int32),
      pltpu.VMEM((NP * STRIDE,), jnp.float32),
      pltpu.VMEM((NP * STRIDE,), jnp.float32),
  ] + [pltpu.SemaphoreType.DMA for _ in range(4)]
  if with_cnt:
    out_type.append(jax.ShapeDtypeStruct((NP,), jnp.float32))
    scratch.append(pltpu.VMEM((NP,), jnp.float32))

  def body(xst_hbm, src_hbm, dst_hbm, z5_hbm, z1_hbm, *refs):
    if with_cnt:
      (out_hbm, cnt_hbm, srcA, dstA, srcB, dstB, table, acc,
       semAs, semAd, semBs, semBd, cacc) = refs
    else:
      (out_hbm, srcA, dstA, srcB, dstB, table, acc,
       semAs, semAd, semBs, semBd) = refs
      cnt_hbm = cacc = None
    slab = lax.axis_index("s") * NC + lax.axis_index("c")

    pltpu.sync_copy(z5_hbm, acc)
    if with_cnt:
      pltpu.sync_copy(z1_hbm, cacc)
    pltpu.sync_copy(xst_hbm.at[slab], table)

    cols = [jnp.full((LANES,), c, jnp.int32) for c in range(SW)]
    ones16 = jnp.ones((LANES,), jnp.float32)

    def accumulate(srcX, dstX, j):
      for k in range(BATCH // LANES):
        s16 = srcX.at[j, pl.ds(k * LANES, LANES)][...] * STRIDE
        d16 = dstX.at[j, pl.ds(k * LANES, LANES)][...]
        d16s = d16 * STRIDE
        for c in range(SW):
          v16 = plsc.load_gather(table, [s16 + cols[c]])
          plsc.addupdate_scatter(acc, [d16s + cols[c]], v16)
        if with_cnt:
          @pl.when(slab == 0)
          def _():
            plsc.addupdate_scatter(cacc, [d16], ones16)

    def start_idx(r, srcX, dstX, semX_s, semX_d):
      pltpu.async_copy(src_hbm.at[pl.ds(r, IDXG)], srcX, semX_s)
      pltpu.async_copy(dst_hbm.at[pl.ds(r, IDXG)], dstX, semX_d)

    def wait_idx(srcX, dstX, semX_s, semX_d):
      pltpu.make_async_copy(src_hbm.at[pl.ds(0, IDXG)], srcX, semX_s).wait()
      pltpu.make_async_copy(dst_hbm.at[pl.ds(0, IDXG)], dstX, semX_d).wait()

    start_idx(0, srcA, dstA, semAs, semAd)
    start_idx(IDXG, srcB, dstB, semBs, semBd)

    @pl.loop(0, NIT)
    def _(i):
      r0 = i * 2 * IDXG
      wait_idx(srcA, dstA, semAs, semAd)
      for j in range(IDXG):
        accumulate(srcA, dstA, j)
      start_idx(r0 + 2 * IDXG, srcA, dstA, semAs, semAd)
      wait_idx(srcB, dstB, semBs, semBd)
      for j in range(IDXG):
        accumulate(srcB, dstB, j)
      start_idx(r0 + 3 * IDXG, srcB, dstB, semBs, semBd)

    wait_idx(srcA, dstA, semAs, semAd)
    wait_idx(srcB, dstB, semBs, semBd)

    pltpu.sync_copy(acc, out_hbm.at[slab])
    if with_cnt:
      @pl.when(slab == 0)
      def _():
        pltpu.sync_copy(cacc, cnt_hbm)

  cp = pltpu.CompilerParams()
  if "needs_layout_passes" in pltpu.CompilerParams.__dataclass_fields__:
    cp = dataclasses.replace(cp, needs_layout_passes=False)
  if "use_tc_tiling_on_sc" in pltpu.CompilerParams.__dataclass_fields__:
    cp = dataclasses.replace(cp, use_tc_tiling_on_sc=False)
  return pl.kernel(body, out_type=out_type, mesh=mesh, scratch_types=scratch,
                   compiler_params=cp)


_agg_cnt = _make_agg(True)
_agg = _make_agg(False)


BM = 512
GRID = NP // BM
CROWS = BM // 128


def _inproj_body(x_ref, w_ref, b_ref, inp_ref, h_ref):
  z = jnp.dot(x_ref[...], w_ref[...],
              preferred_element_type=jnp.float32) + b_ref[...]
  inp_ref[...] = z
  h_ref[...] = jnp.maximum(z, 0.0)


_inproj = pl.pallas_call(
    _inproj_body,
    grid=(GRID,),
    in_specs=[
        pl.BlockSpec((BM, H), lambda i: (i, 0)),
        pl.BlockSpec((H, H), lambda i: (0, 0)),
        pl.BlockSpec((1, H), lambda i: (0, 0)),
    ],
    out_specs=[
        pl.BlockSpec((BM, H), lambda i: (i, 0)),
        pl.BlockSpec((BM, H), lambda i: (i, 0)),
    ],
    out_shape=[jax.ShapeDtypeStruct((NP, H), jnp.float32)] * 2,
)


def _combine_body(final, parts_ref, cnts_ref, h_ref, inp_ref, wl_ref, bl_ref,
                  wr_ref, o_ref):
  agg = parts_ref[...]
  cnt = cnts_ref[...]
  agg = agg / jnp.maximum(cnt, 1.0)
  z = (jnp.dot(agg, wl_ref[...], preferred_element_type=jnp.float32)
       + bl_ref[...]
       + jnp.dot(h_ref[...], wr_ref[...], preferred_element_type=jnp.float32))
  if final:
    m = jnp.max(z, axis=-1, keepdims=True)
    lse = jnp.log(jnp.sum(jnp.exp(z - m), axis=-1, keepdims=True)) + m
    o_ref[...] = z - lse
  else:
    o_ref[...] = jnp.maximum(z, 0.0) + 0.2 * inp_ref[...]


def _make_combine(final):
  return pl.pallas_call(
      functools.partial(_combine_body, final),
      grid=(GRID,),
      in_specs=[
          pl.BlockSpec((BM, H), lambda i: (i, 0)),
          pl.BlockSpec((BM, 1), lambda i: (i, 0)),
          pl.BlockSpec((BM, H), lambda i: (i, 0)),
          pl.BlockSpec((BM, H), lambda i: (i, 0)),
          pl.BlockSpec((H, H), lambda i: (0, 0)),
          pl.BlockSpec((1, H), lambda i: (0, 0)),
          pl.BlockSpec((H, H), lambda i: (0, 0)),
      ],
      out_specs=pl.BlockSpec((BM, H), lambda i: (i, 0)),
      out_shape=jax.ShapeDtypeStruct((NP, H), jnp.float32),
  )


_combine_mid = _make_combine(False)
_combine_final = _make_combine(True)


def kernel(x, edge_index, W_in, b_in, Wl0, bl0, Wr0, Wl1, bl1, Wr1,
           Wl2, bl2, Wr2):
  pad = IDX_ROWS * BATCH - E
  src = jnp.concatenate(
      [edge_index[0], jnp.zeros((pad,), jnp.int32)]).reshape(IDX_ROWS, BATCH)
  dst = jnp.concatenate(
      [edge_index[1], jnp.full((pad,), N, jnp.int32)]).reshape(IDX_ROWS, BATCH)
  x_pad = jnp.concatenate([x, jnp.zeros((NP - N, H), jnp.float32)])

  z5 = jnp.zeros((NP * STRIDE,), jnp.float32)
  z1 = jnp.zeros((NP,), jnp.float32)

  def slabbed(a):
    a = a.reshape(NP, SLABS, SW)
    a = jnp.pad(a, ((0, 0), (0, 0), (0, STRIDE - SW)))
    return a.transpose(1, 0, 2).reshape(SLABS, NP * STRIDE)

  def unslabbed(p):
    p = p.reshape(SLABS, NP, STRIDE)[:, :, :SW]
    return p.transpose(1, 0, 2).reshape(NP, H)

  inp, h = _inproj(x_pad, W_in.T, b_in.reshape(1, H))

  parts, cnts = _agg_cnt(slabbed(h), src, dst, z5, z1)
  cnt3 = cnts.reshape(NP, 1)
  h = _combine_mid(unslabbed(parts), cnt3, h, inp, Wl0.T,
                   bl0.reshape(1, H), Wr0.T)
  parts, = _agg(slabbed(h), src, dst, z5, z1)
  h = _combine_mid(unslabbed(parts), cnt3, h, inp, Wl1.T,
                   bl1.reshape(1, H), Wr1.T)
  parts, = _agg(slabbed(h), src, dst, z5, z1)
  out = _combine_final(unslabbed(parts), cnt3, h, inp, Wl2.T,
                       bl2.reshape(1, H), Wr2.T)
  return out[:N]
